# baseline (speedup 1.0000x reference)
import functools

import jax
import jax.numpy as jnp
from jax import lax
from jax.experimental import pallas as pl
from jax.experimental.pallas import tpu as pltpu
from jax.experimental.pallas import tpu_sc as plsc

H = 128
GRIDS = (8, 8, 8)

NC = 2
NS = 16
LANES = 16
WSUB = 32
NPH = 4


def _ln(x):
    mu = jnp.mean(x, axis=-1, keepdims=True)
    var = jnp.var(x, axis=-1, keepdims=True)
    return (x - mu) / jnp.sqrt(var + 1e-5)


@functools.partial(jax.jit, static_argnames=("S", "E", "CH"))
def _segsum_cols_sc(values4, idx2d, *, S, E, CH=1024):
    assert E % CH == 0 and CH % 128 == 0
    nchunks = E // CH
    CHR = CH // 128
    S = -(-S // (NS * 8)) * (NS * 8)
    rps = S // NS
    assert rps <= CH

    mesh = plsc.VectorSubcoreMesh(core_axis_name="c", subcore_axis_name="s")

    @functools.partial(
        pl.kernel,
        out_type=jax.ShapeDtypeStruct((NPH, S, WSUB), jnp.float32),
        mesh=mesh,
        scratch_types=[
            pltpu.VMEM((CHR, 128), jnp.int32),
            pltpu.VMEM((CH, WSUB), jnp.float32),
            pltpu.VMEM((rps, WSUB), jnp.float32),
            pltpu.VMEM_SHARED((S, WSUB), jnp.float32),
        ],
        compiler_params=pltpu.CompilerParams(use_tc_tiling_on_sc=False),
    )
    def k(vh, idxh, outh, idx_v, vals_v, zbuf, shared):
        c = lax.axis_index("c")
        s = lax.axis_index("s")

        def zrow(r, _):
            for j in range(WSUB // LANES):
                zbuf[r, pl.ds(j * LANES, LANES)] = jnp.zeros(
                    (LANES,), jnp.float32)
            return 0
        lax.fori_loop(0, rps, zrow, 0)
        pltpu.sync_copy(zbuf, shared.at[pl.ds(s * rps, rps)])
        plsc.subcore_barrier()

        my_n = (nchunks - s + NS - 1) // NS

        for p in range(NPH):
            n_p = jnp.where(c == p // 2, my_n, 0)

            def body(i, _, p=p):
                k0 = s + i * NS
                pltpu.sync_copy(idxh.at[pl.ds(k0 * CHR, CHR)], idx_v)
                pltpu.sync_copy(vh.at[p, pl.ds(k0 * CH, CH)], vals_v)
                for u in range(CHR):
                    pltpu.sync_copy(
                        vals_v.at[pl.ds(u * 128, 128)],
                        shared.at[idx_v.at[u]], add=True)
                return 0
            lax.fori_loop(0, n_p, body, 0)
            plsc.subcore_barrier()

            @pl.when(c == p // 2)
            def _drain(p=p):
                pltpu.sync_copy(shared.at[pl.ds(s * rps, rps)],
                                vals_v.at[pl.ds(0, rps)])
                pltpu.sync_copy(vals_v.at[pl.ds(0, rps)],
                                outh.at[p, pl.ds(s * rps, rps)])
                pltpu.sync_copy(zbuf, shared.at[pl.ds(s * rps, rps)])
            plsc.subcore_barrier()

    return k(values4, idx2d)


def _segsum(values, idx, S):
    E = values.shape[0]
    EP = -(-E // 1024) * 1024
    if EP != E:
        values = jnp.pad(values, ((0, EP - E), (0, 0)))
        idx = jnp.pad(idx, (0, EP - E))
    values4 = jnp.transpose(values.reshape(EP, NPH, WSUB), (1, 0, 2))
    idx2d = idx.reshape(EP // 128, 128)
    out = _segsum_cols_sc(values4, idx2d, S=S, E=EP)
    return jnp.concatenate([out[p] for p in range(NPH)], axis=-1)[:S]


def kernel(a_x, m_x, m, rbf3, cbf3, id3_ragged_idx, id_swap, id3_ba, id3_ca, rbf_h, idx_s, idx_t, a2m_edge_index, m2a_edge_index, a2m_edge_weights, m2a_edge_weights, a2m_edge_attr, m2a_edge_attr, W_rbf, W_cbf, W_h, W_e2a, W_attr_a2m, W_out_a2m, W_attr_m2a, W_out_m2a, W_combine, b_combine, conv_w, conv_b):
    delta_m_x = m_x
    a = _ln(a_x)
    me = _ln(m)
    x_ba = me[id3_ba]
    x3 = x_ba * (rbf3 @ W_rbf) * (cbf3 @ W_cbf)
    m2 = me + jax.ops.segment_sum(x3, id3_ca, num_segments=me.shape[0])
    m2 = m2 + m2[id_swap]
    gate = jax.nn.sigmoid(rbf_h @ W_h)
    a_agg = _segsum(m2 * gate, idx_t, a.shape[0])
    a2 = a + a_agg @ W_e2a
    mx = _ln(m_x)
    B = mx.shape[0] // (GRIDS[0] * GRIDS[1] * GRIDS[2])
    g = mx.reshape(B, GRIDS[0], GRIDS[1], GRIDS[2], H).transpose(0, 4, 1, 2, 3)
    g = jax.lax.conv_general_dilated(g, conv_w, (1, 1, 1), 'SAME',
                                     dimension_numbers=('NCDHW', 'OIDHW', 'NCDHW'))
    g = g + conv_b[None, :, None, None, None]
    mx2 = g.transpose(0, 2, 3, 4, 1).reshape(-1, H)
    src_a, dst_m = a2m_edge_index[0], a2m_edge_index[1]
    msg_a2m = a2[src_a] * a2m_edge_weights[:, None] + a2m_edge_attr @ W_attr_a2m
    a2m_message = _segsum(msg_a2m, dst_m, mx2.shape[0]) @ W_out_a2m
    a2m_message = _ln(a2m_message)
    src_m, dst_a = m2a_edge_index[0], m2a_edge_index[1]
    msg_m2a = mx2[src_m] * m2a_edge_weights[:, None] + m2a_edge_attr @ W_attr_m2a
    m2a_message = _segsum(msg_m2a, dst_a, a2.shape[0]) @ W_out_m2a
    m2a_j = m2a_message[idx_s]
    m2a_i = m2a_message[idx_t]
    edge_msg = jax.nn.silu(jnp.concatenate([m2a_j, m2a_i], axis=-1) @ W_combine + b_combine)
    edge_msg = _ln(edge_msg)
    return (a2, mx2 + a2m_message + delta_m_x, m2 + edge_msg)


# speedup vs baseline: 1.1565x; 1.1565x over previous
import functools

import jax
import jax.numpy as jnp
from jax import lax
from jax.experimental import pallas as pl
from jax.experimental.pallas import tpu as pltpu
from jax.experimental.pallas import tpu_sc as plsc

H = 128
GRIDS = (8, 8, 8)

NC = 2
NS = 16
LANES = 16
WSUB = 32
NPH = 4


def _ln(x):
    mu = jnp.mean(x, axis=-1, keepdims=True)
    var = jnp.var(x, axis=-1, keepdims=True)
    return (x - mu) / jnp.sqrt(var + 1e-5)


@functools.partial(jax.jit, static_argnames=("S", "E", "CH"))
def _segsum_cols_sc(values, idx2d, *, S, E, CH=512):
    assert E % CH == 0 and CH % 128 == 0
    nchunks = E // CH
    CHR = CH // 128
    S = -(-S // (NS * 8)) * (NS * 8)
    rps = S // NS

    mesh = plsc.VectorSubcoreMesh(core_axis_name="c", subcore_axis_name="s")

    @functools.partial(
        pl.kernel,
        out_type=jax.ShapeDtypeStruct((NPH, S, WSUB), jnp.float32),
        mesh=mesh,
        scratch_types=[
            pltpu.VMEM((CHR, 128), jnp.int32),
            pltpu.VMEM((CH, WSUB), jnp.float32),
            pltpu.VMEM((rps, WSUB), jnp.float32),
            pltpu.VMEM((rps, WSUB), jnp.float32),
            pltpu.VMEM_SHARED((S, WSUB), jnp.float32),
        ],
        compiler_params=pltpu.CompilerParams(use_tc_tiling_on_sc=False),
    )
    def k(vh, idxh, outh, idx_v, vals_v, zbuf, obuf, shared):
        c = lax.axis_index("c")
        s = lax.axis_index("s")

        def zrow(r, _):
            for j in range(WSUB // LANES):
                zbuf[r, pl.ds(j * LANES, LANES)] = jnp.zeros(
                    (LANES,), jnp.float32)
            return 0
        lax.fori_loop(0, rps, zrow, 0)
        pltpu.sync_copy(zbuf, shared.at[pl.ds(s * rps, rps)])
        plsc.subcore_barrier()

        my_n = (nchunks - s + NS - 1) // NS

        for p in range(NPH):
            n_p = jnp.where(c == p // 2, my_n, 0)

            def body(i, _, p=p):
                k0 = s + i * NS
                pltpu.sync_copy(idxh.at[pl.ds(k0 * CHR, CHR)], idx_v)
                pltpu.sync_copy(
                    vh.at[pl.ds(k0 * CH, CH), pl.ds(p * WSUB, WSUB)], vals_v)
                for u in range(CHR):
                    pltpu.sync_copy(
                        vals_v.at[pl.ds(u * 128, 128)],
                        shared.at[idx_v.at[u]], add=True)
                return 0
            lax.fori_loop(0, n_p, body, 0)
            plsc.subcore_barrier()

            @pl.when(c == p // 2)
            def _drain(p=p):
                pltpu.sync_copy(shared.at[pl.ds(s * rps, rps)], obuf)
                pltpu.sync_copy(obuf, outh.at[p, pl.ds(s * rps, rps)])
                pltpu.sync_copy(zbuf, shared.at[pl.ds(s * rps, rps)])
            plsc.subcore_barrier()

    return k(values, idx2d)


def _segsum(values, idx, S):
    E = values.shape[0]
    assert E % 512 == 0
    idx2d = idx.reshape(E // 128, 128)
    out = _segsum_cols_sc(values, idx2d, S=S, E=E)
    return jnp.concatenate([out[p] for p in range(NPH)], axis=-1)[:S]


def kernel(a_x, m_x, m, rbf3, cbf3, id3_ragged_idx, id_swap, id3_ba, id3_ca, rbf_h, idx_s, idx_t, a2m_edge_index, m2a_edge_index, a2m_edge_weights, m2a_edge_weights, a2m_edge_attr, m2a_edge_attr, W_rbf, W_cbf, W_h, W_e2a, W_attr_a2m, W_out_a2m, W_attr_m2a, W_out_m2a, W_combine, b_combine, conv_w, conv_b):
    delta_m_x = m_x
    a = _ln(a_x)
    me = _ln(m)
    x_ba = me[id3_ba]
    x3 = x_ba * (rbf3 @ W_rbf) * (cbf3 @ W_cbf)
    m2 = me + jax.ops.segment_sum(x3, id3_ca, num_segments=me.shape[0])
    m2 = m2 + m2[id_swap]
    gate = jax.nn.sigmoid(rbf_h @ W_h)
    a_agg = _segsum(m2 * gate, idx_t, a.shape[0])
    a2 = a + a_agg @ W_e2a
    mx = _ln(m_x)
    B = mx.shape[0] // (GRIDS[0] * GRIDS[1] * GRIDS[2])
    g = mx.reshape(B, GRIDS[0], GRIDS[1], GRIDS[2], H).transpose(0, 4, 1, 2, 3)
    g = jax.lax.conv_general_dilated(g, conv_w, (1, 1, 1), 'SAME',
                                     dimension_numbers=('NCDHW', 'OIDHW', 'NCDHW'))
    g = g + conv_b[None, :, None, None, None]
    mx2 = g.transpose(0, 2, 3, 4, 1).reshape(-1, H)
    EB = a2m_edge_index.shape[1]
    EBP = -(-EB // 512) * 512
    pe = EBP - EB
    src_a, dst_m = a2m_edge_index[0], a2m_edge_index[1]
    src_a, dst_m = jnp.pad(src_a, (0, pe)), jnp.pad(dst_m, (0, pe))
    w_a2m = jnp.pad(a2m_edge_weights, (0, pe))
    src_m, dst_a = m2a_edge_index[0], m2a_edge_index[1]
    src_m, dst_a = jnp.pad(src_m, (0, pe)), jnp.pad(dst_a, (0, pe))
    w_m2a = jnp.pad(m2a_edge_weights, (0, pe))
    agg_a2m = _segsum(a2[src_a] * w_a2m[:, None], dst_m, mx2.shape[0])
    attr_agg_a2m = jax.ops.segment_sum(a2m_edge_attr, a2m_edge_index[1], num_segments=mx2.shape[0])
    a2m_message = (agg_a2m + attr_agg_a2m @ W_attr_a2m) @ W_out_a2m
    a2m_message = _ln(a2m_message)
    agg_m2a = _segsum(mx2[src_m] * w_m2a[:, None], dst_a, a2.shape[0])
    attr_agg_m2a = jax.ops.segment_sum(m2a_edge_attr, m2a_edge_index[1], num_segments=a2.shape[0])
    m2a_message = (agg_m2a + attr_agg_m2a @ W_attr_m2a) @ W_out_m2a
    P1 = m2a_message @ W_combine[:H]
    P2 = m2a_message @ W_combine[H:]
    edge_msg = jax.nn.silu(P1[idx_s] + P2[idx_t] + b_combine)
    edge_msg = _ln(edge_msg)
    return (a2, mx2 + a2m_message + delta_m_x, m2 + edge_msg)


# speedup vs baseline: 1.2617x; 1.0910x over previous
import functools

import jax
import jax.numpy as jnp
from jax import lax
from jax.experimental import pallas as pl
from jax.experimental.pallas import tpu as pltpu
from jax.experimental.pallas import tpu_sc as plsc

H = 128
GRIDS = (8, 8, 8)

NC = 2
NS = 16
LANES = 16
WSUB = 32
NPH = 4


def _ln(x):
    mu = jnp.mean(x, axis=-1, keepdims=True)
    var = jnp.var(x, axis=-1, keepdims=True)
    return (x - mu) / jnp.sqrt(var + 1e-5)


@functools.partial(jax.jit, static_argnames=("S", "E", "CH"))
def _segsum_cols_sc(values, idx2d, *, S, E, CH=512):
    assert E % CH == 0 and CH % 128 == 0
    nchunks = E // CH
    CHR = CH // 128
    S = -(-S // (NS * 8)) * (NS * 8)
    rps = S // NS

    mesh = plsc.VectorSubcoreMesh(core_axis_name="c", subcore_axis_name="s")

    @functools.partial(
        pl.kernel,
        out_type=jax.ShapeDtypeStruct((NPH, S, WSUB), jnp.float32),
        mesh=mesh,
        scratch_types=[
            pltpu.VMEM((CHR, 128), jnp.int32),
            pltpu.VMEM((CH, WSUB), jnp.float32),
            pltpu.VMEM((rps, WSUB), jnp.float32),
            pltpu.VMEM((rps, WSUB), jnp.float32),
            pltpu.VMEM_SHARED((S, WSUB), jnp.float32),
        ],
        compiler_params=pltpu.CompilerParams(use_tc_tiling_on_sc=False),
    )
    def k(vh, idxh, outh, idx_v, vals_v, zbuf, obuf, shared):
        c = lax.axis_index("c")
        s = lax.axis_index("s")

        def zrow(r, _):
            for j in range(WSUB // LANES):
                zbuf[r, pl.ds(j * LANES, LANES)] = jnp.zeros(
                    (LANES,), jnp.float32)
            return 0
        lax.fori_loop(0, rps, zrow, 0)
        pltpu.sync_copy(zbuf, shared.at[pl.ds(s * rps, rps)])
        plsc.subcore_barrier()

        my_n = (nchunks - s + NS - 1) // NS

        for p in range(NPH):
            n_p = jnp.where(c == p // 2, my_n, 0)

            def body(i, _, p=p):
                k0 = s + i * NS
                pltpu.sync_copy(idxh.at[pl.ds(k0 * CHR, CHR)], idx_v)
                pltpu.sync_copy(
                    vh.at[pl.ds(k0 * CH, CH), pl.ds(p * WSUB, WSUB)], vals_v)
                for u in range(CHR):
                    pltpu.sync_copy(
                        vals_v.at[pl.ds(u * 128, 128)],
                        shared.at[idx_v.at[u]], add=True)
                return 0
            lax.fori_loop(0, n_p, body, 0)
            plsc.subcore_barrier()

            @pl.when(c == p // 2)
            def _drain(p=p):
                pltpu.sync_copy(shared.at[pl.ds(s * rps, rps)], obuf)
                pltpu.sync_copy(obuf, outh.at[p, pl.ds(s * rps, rps)])
                pltpu.sync_copy(zbuf, shared.at[pl.ds(s * rps, rps)])
            plsc.subcore_barrier()

    return k(values, idx2d)


def _segsum(values, idx, S):
    E = values.shape[0]
    assert E % 512 == 0
    idx2d = idx.reshape(E // 128, 128)
    out = _segsum_cols_sc(values, idx2d, S=S, E=E)
    return jnp.concatenate([out[p] for p in range(NPH)], axis=-1)[:S]


@functools.partial(jax.jit, static_argnames=("S", "E", "CH"))
def _segsum16_sc(values, idx2d, *, S, E, CH=512):
    assert E % CH == 0 and CH % 128 == 0
    nchunks = E // CH
    CHR = CH // 128
    S = -(-S // (NS * 8)) * (NS * 8)
    rps = S // NS

    mesh = plsc.VectorSubcoreMesh(core_axis_name="c", subcore_axis_name="s")

    @functools.partial(
        pl.kernel,
        out_type=jax.ShapeDtypeStruct((NC, S, 16), jnp.float32),
        mesh=mesh,
        scratch_types=[
            pltpu.VMEM((CHR, 128), jnp.int32),
            pltpu.VMEM((CH, 16), jnp.float32),
            pltpu.VMEM((rps, 16), jnp.float32),
            pltpu.VMEM_SHARED((S, 16), jnp.float32),
        ],
        compiler_params=pltpu.CompilerParams(use_tc_tiling_on_sc=False),
    )
    def k(vh, idxh, outh, idx_v, vals_v, zbuf, shared):
        c = lax.axis_index("c")
        s = lax.axis_index("s")
        w = s * NC + c

        def zrow(r, _):
            zbuf[r, pl.ds(0, LANES)] = jnp.zeros((LANES,), jnp.float32)
            return 0
        lax.fori_loop(0, rps, zrow, 0)
        pltpu.sync_copy(zbuf, shared.at[pl.ds(s * rps, rps)])
        plsc.subcore_barrier()

        my_n = (nchunks - w + NC * NS - 1) // (NC * NS)

        def body(i, _):
            k0 = w + i * NC * NS
            pltpu.sync_copy(idxh.at[pl.ds(k0 * CHR, CHR)], idx_v)
            pltpu.sync_copy(vh.at[pl.ds(k0 * CH, CH)], vals_v)
            for u in range(CHR):
                pltpu.sync_copy(vals_v.at[pl.ds(u * 128, 128)],
                                shared.at[idx_v.at[u]], add=True)
            return 0
        lax.fori_loop(0, my_n, body, 0)
        plsc.subcore_barrier()
        pltpu.sync_copy(shared.at[pl.ds(s * rps, rps)], zbuf)
        pltpu.sync_copy(zbuf, outh.at[c, pl.ds(s * rps, rps)])

    return k(values, idx2d)


def _segsum16(values, idx, S):
    E = values.shape[0]
    assert E % 512 == 0 and values.shape[1] == 16
    idx2d = idx.reshape(E // 128, 128)
    out = _segsum16_sc(values, idx2d, S=S, E=E)
    return (out[0] + out[1])[:S]


def kernel(a_x, m_x, m, rbf3, cbf3, id3_ragged_idx, id_swap, id3_ba, id3_ca, rbf_h, idx_s, idx_t, a2m_edge_index, m2a_edge_index, a2m_edge_weights, m2a_edge_weights, a2m_edge_attr, m2a_edge_attr, W_rbf, W_cbf, W_h, W_e2a, W_attr_a2m, W_out_a2m, W_attr_m2a, W_out_m2a, W_combine, b_combine, conv_w, conv_b):
    delta_m_x = m_x
    a = _ln(a_x)
    me = _ln(m)
    x_ba = me[id3_ba]
    x3 = x_ba * (rbf3 @ W_rbf) * (cbf3 @ W_cbf)
    m2 = me + jax.ops.segment_sum(x3, id3_ca, num_segments=me.shape[0])
    m2 = m2 + m2[id_swap]
    gate = jax.nn.sigmoid(rbf_h @ W_h)
    a_agg = _segsum(m2 * gate, idx_t, a.shape[0])
    a2 = a + a_agg @ W_e2a
    mx = _ln(m_x)
    B = mx.shape[0] // (GRIDS[0] * GRIDS[1] * GRIDS[2])
    g = mx.reshape(B, GRIDS[0], GRIDS[1], GRIDS[2], H).transpose(0, 4, 1, 2, 3)
    g = jax.lax.conv_general_dilated(g, conv_w, (1, 1, 1), 'SAME',
                                     dimension_numbers=('NCDHW', 'OIDHW', 'NCDHW'))
    g = g + conv_b[None, :, None, None, None]
    mx2 = g.transpose(0, 2, 3, 4, 1).reshape(-1, H)
    EB = a2m_edge_index.shape[1]
    EBP = -(-EB // 512) * 512
    pe = EBP - EB
    src_a, dst_m = a2m_edge_index[0], a2m_edge_index[1]
    src_a, dst_m = jnp.pad(src_a, (0, pe)), jnp.pad(dst_m, (0, pe))
    w_a2m = jnp.pad(a2m_edge_weights, (0, pe))
    src_m, dst_a = m2a_edge_index[0], m2a_edge_index[1]
    src_m, dst_a = jnp.pad(src_m, (0, pe)), jnp.pad(dst_a, (0, pe))
    w_m2a = jnp.pad(m2a_edge_weights, (0, pe))
    attr_a2m = jnp.pad(a2m_edge_attr, ((0, pe), (0, 0)))
    attr_m2a = jnp.pad(m2a_edge_attr, ((0, pe), (0, 0)))
    agg_a2m = _segsum(a2[src_a] * w_a2m[:, None], dst_m, mx2.shape[0])
    attr_agg_a2m = _segsum16(attr_a2m, dst_m, mx2.shape[0])
    a2m_message = (agg_a2m + attr_agg_a2m @ W_attr_a2m) @ W_out_a2m
    a2m_message = _ln(a2m_message)
    agg_m2a = _segsum(mx2[src_m] * w_m2a[:, None], dst_a, a2.shape[0])
    attr_agg_m2a = _segsum16(attr_m2a, dst_a, a2.shape[0])
    m2a_message = (agg_m2a + attr_agg_m2a @ W_attr_m2a) @ W_out_m2a
    P1 = m2a_message @ W_combine[:H]
    P2 = m2a_message @ W_combine[H:]
    edge_msg = jax.nn.silu(P1[idx_s] + P2[idx_t] + b_combine)
    edge_msg = _ln(edge_msg)
    return (a2, mx2 + a2m_message + delta_m_x, m2 + edge_msg)


# speedup vs baseline: 1.2619x; 1.0001x over previous
import functools

import jax
import jax.numpy as jnp
from jax import lax
from jax.experimental import pallas as pl
from jax.experimental.pallas import tpu as pltpu
from jax.experimental.pallas import tpu_sc as plsc

H = 128
GRIDS = (8, 8, 8)

NC = 2
NS = 16
LANES = 16
WSUB = 32
NPH = 4


def _ln(x):
    mu = jnp.mean(x, axis=-1, keepdims=True)
    var = jnp.var(x, axis=-1, keepdims=True)
    return (x - mu) / jnp.sqrt(var + 1e-5)


@functools.partial(jax.jit, static_argnames=("S", "E", "CH"))
def _segsum_cols_sc(values, idx2d, *, S, E, CH=512):
    assert E % CH == 0 and CH % 128 == 0
    nchunks = E // CH
    CHR = CH // 128
    S = -(-S // (NS * 8)) * (NS * 8)
    rps = S // NS

    mesh = plsc.VectorSubcoreMesh(core_axis_name="c", subcore_axis_name="s")

    @functools.partial(
        pl.kernel,
        out_type=jax.ShapeDtypeStruct((NPH, S, WSUB), jnp.float32),
        mesh=mesh,
        scratch_types=[
            pltpu.VMEM((CHR, 128), jnp.int32),
            pltpu.VMEM((CH, WSUB), jnp.float32),
            pltpu.VMEM((rps, WSUB), jnp.float32),
            pltpu.VMEM((rps, WSUB), jnp.float32),
            pltpu.VMEM_SHARED((S, WSUB), jnp.float32),
        ],
        compiler_params=pltpu.CompilerParams(use_tc_tiling_on_sc=False),
    )
    def k(vh, idxh, outh, idx_v, vals_v, zbuf, obuf, shared):
        c = lax.axis_index("c")
        s = lax.axis_index("s")

        def zrow(r, _):
            for j in range(WSUB // LANES):
                zbuf[r, pl.ds(j * LANES, LANES)] = jnp.zeros(
                    (LANES,), jnp.float32)
            return 0
        lax.fori_loop(0, rps, zrow, 0)
        pltpu.sync_copy(zbuf, shared.at[pl.ds(s * rps, rps)])
        plsc.subcore_barrier()

        my_n = (nchunks - s + NS - 1) // NS

        for p in range(NPH):
            n_p = jnp.where(c == p // 2, my_n, 0)

            def body(i, _, p=p):
                k0 = s + i * NS
                pltpu.sync_copy(idxh.at[pl.ds(k0 * CHR, CHR)], idx_v)
                pltpu.sync_copy(
                    vh.at[pl.ds(k0 * CH, CH), pl.ds(p * WSUB, WSUB)], vals_v)
                for u in range(CHR):
                    pltpu.sync_copy(
                        vals_v.at[pl.ds(u * 128, 128)],
                        shared.at[idx_v.at[u]], add=True)
                return 0
            lax.fori_loop(0, n_p, body, 0)
            plsc.subcore_barrier()

            @pl.when(c == p // 2)
            def _drain(p=p):
                pltpu.sync_copy(shared.at[pl.ds(s * rps, rps)], obuf)
                pltpu.sync_copy(obuf, outh.at[p, pl.ds(s * rps, rps)])
                pltpu.sync_copy(zbuf, shared.at[pl.ds(s * rps, rps)])
            plsc.subcore_barrier()

    return k(values, idx2d)


def _segsum(values, idx, S):
    E = values.shape[0]
    assert E % 512 == 0
    idx2d = idx.reshape(E // 128, 128)
    out = _segsum_cols_sc(values, idx2d, S=S, E=E)
    return jnp.concatenate([out[p] for p in range(NPH)], axis=-1)[:S]


@functools.partial(jax.jit, static_argnames=("S", "E", "CH"))
def _segsum16_sc(values, idx2d, *, S, E, CH=512):
    assert E % CH == 0 and CH % 128 == 0
    nchunks = E // CH
    CHR = CH // 128
    S = -(-S // (NS * 8)) * (NS * 8)
    rps = S // NS

    mesh = plsc.VectorSubcoreMesh(core_axis_name="c", subcore_axis_name="s")

    @functools.partial(
        pl.kernel,
        out_type=jax.ShapeDtypeStruct((NC, S, 16), jnp.float32),
        mesh=mesh,
        scratch_types=[
            pltpu.VMEM((CHR, 128), jnp.int32),
            pltpu.VMEM((CH, 16), jnp.float32),
            pltpu.VMEM((rps, 16), jnp.float32),
            pltpu.VMEM_SHARED((S, 16), jnp.float32),
        ],
        compiler_params=pltpu.CompilerParams(use_tc_tiling_on_sc=False),
    )
    def k(vh, idxh, outh, idx_v, vals_v, zbuf, shared):
        c = lax.axis_index("c")
        s = lax.axis_index("s")
        w = s * NC + c

        def zrow(r, _):
            zbuf[r, pl.ds(0, LANES)] = jnp.zeros((LANES,), jnp.float32)
            return 0
        lax.fori_loop(0, rps, zrow, 0)
        pltpu.sync_copy(zbuf, shared.at[pl.ds(s * rps, rps)])
        plsc.subcore_barrier()

        my_n = (nchunks - w + NC * NS - 1) // (NC * NS)

        def body(i, _):
            k0 = w + i * NC * NS
            pltpu.sync_copy(idxh.at[pl.ds(k0 * CHR, CHR)], idx_v)
            pltpu.sync_copy(vh.at[pl.ds(k0 * CH, CH)], vals_v)
            for u in range(CHR):
                pltpu.sync_copy(vals_v.at[pl.ds(u * 128, 128)],
                                shared.at[idx_v.at[u]], add=True)
            return 0
        lax.fori_loop(0, my_n, body, 0)
        plsc.subcore_barrier()
        pltpu.sync_copy(shared.at[pl.ds(s * rps, rps)], zbuf)
        pltpu.sync_copy(zbuf, outh.at[c, pl.ds(s * rps, rps)])

    return k(values, idx2d)


def _segsum16(values, idx, S):
    E = values.shape[0]
    assert E % 512 == 0 and values.shape[1] == 16
    idx2d = idx.reshape(E // 128, 128)
    out = _segsum16_sc(values, idx2d, S=S, E=E)
    return (out[0] + out[1])[:S]


def kernel(a_x, m_x, m, rbf3, cbf3, id3_ragged_idx, id_swap, id3_ba, id3_ca, rbf_h, idx_s, idx_t, a2m_edge_index, m2a_edge_index, a2m_edge_weights, m2a_edge_weights, a2m_edge_attr, m2a_edge_attr, W_rbf, W_cbf, W_h, W_e2a, W_attr_a2m, W_out_a2m, W_attr_m2a, W_out_m2a, W_combine, b_combine, conv_w, conv_b):
    delta_m_x = m_x
    a = _ln(a_x)
    me = _ln(m)
    x_ba = me[id3_ba]
    x3 = x_ba * (rbf3 @ W_rbf) * (cbf3 @ W_cbf)
    m2 = me.at[id3_ca].add(x3)
    m2 = m2 + m2[id_swap]
    gate = jax.nn.sigmoid(rbf_h @ W_h)
    a_agg = _segsum(m2 * gate, idx_t, a.shape[0])
    a2 = a + a_agg @ W_e2a
    mx = _ln(m_x)
    B = mx.shape[0] // (GRIDS[0] * GRIDS[1] * GRIDS[2])
    g = mx.reshape(B, GRIDS[0], GRIDS[1], GRIDS[2], H).transpose(0, 4, 1, 2, 3)
    g = jax.lax.conv_general_dilated(g, conv_w, (1, 1, 1), 'SAME',
                                     dimension_numbers=('NCDHW', 'OIDHW', 'NCDHW'))
    g = g + conv_b[None, :, None, None, None]
    mx2 = g.transpose(0, 2, 3, 4, 1).reshape(-1, H)
    EB = a2m_edge_index.shape[1]
    EBP = -(-EB // 512) * 512
    pe = EBP - EB
    src_a, dst_m = a2m_edge_index[0], a2m_edge_index[1]
    src_a, dst_m = jnp.pad(src_a, (0, pe)), jnp.pad(dst_m, (0, pe))
    w_a2m = jnp.pad(a2m_edge_weights, (0, pe))
    src_m, dst_a = m2a_edge_index[0], m2a_edge_index[1]
    src_m, dst_a = jnp.pad(src_m, (0, pe)), jnp.pad(dst_a, (0, pe))
    w_m2a = jnp.pad(m2a_edge_weights, (0, pe))
    attr_a2m = jnp.pad(a2m_edge_attr, ((0, pe), (0, 0)))
    attr_m2a = jnp.pad(m2a_edge_attr, ((0, pe), (0, 0)))
    agg_a2m = _segsum(a2[src_a] * w_a2m[:, None], dst_m, mx2.shape[0])
    attr_agg_a2m = _segsum16(attr_a2m, dst_m, mx2.shape[0])
    a2m_message = (agg_a2m + attr_agg_a2m @ W_attr_a2m) @ W_out_a2m
    a2m_message = _ln(a2m_message)
    agg_m2a = _segsum(mx2[src_m] * w_m2a[:, None], dst_a, a2.shape[0])
    attr_agg_m2a = _segsum16(attr_m2a, dst_a, a2.shape[0])
    m2a_message = (agg_m2a + attr_agg_m2a @ W_attr_m2a) @ W_out_m2a
    P1 = m2a_message @ W_combine[:H]
    P2 = m2a_message @ W_combine[H:]
    edge_msg = jax.nn.silu(P1[idx_s] + P2[idx_t] + b_combine)
    edge_msg = _ln(edge_msg)
    return (a2, mx2 + a2m_message + delta_m_x, m2 + edge_msg)
